# baseline (speedup 1.0000x reference)
import functools

import jax
import jax.numpy as jnp
from jax.experimental import pallas as pl
from jax.experimental.pallas import tpu as pltpu

NUM_SELECTS = 8
NOISE_EPS = 0.01
BLW = 0.01
_NEG_BIG = -3.0e38
_INV_SQRT2 = 0.7071067811865476


def _ndtr(z):
    return 0.5 * (1.0 + jax.lax.erf(z * _INV_SQRT2))


def _router_body(x_ref, wc_ref, w2_ref, noise_ref,
                 idx_out, scores_out, loss_out, load_out, imp_out):
    i = pl.program_id(0)
    n = pl.num_programs(0)
    blk, e = noise_ref.shape

    mm = jax.lax.dot_general(
        wc_ref[...], x_ref[...], (((1,), (1,)), ((), ())),
        preferred_element_type=jnp.float32,
        precision=jax.lax.Precision.DEFAULT)
    h = jnp.tanh(mm[:e, :])
    noise_mm = mm[e:, :]
    logits_gate = jax.lax.dot_general(
        w2_ref[...], h, (((1,), (0,)), ((), ())),
        preferred_element_type=jnp.float32,
        precision=jax.lax.Precision.DEFAULT)
    noise_control = (jnp.maximum(noise_mm, 0.0)
                     + jnp.log1p(jnp.exp(-jnp.abs(noise_mm))) + NOISE_EPS)
    logits_noise = noise_ref[...].T * noise_control
    logits = logits_gate + logits_noise

    rf = ((e - 1) - jax.lax.broadcasted_iota(jnp.int32, (e, blk), 0)
          ).astype(jnp.float32)
    work = logits
    sel_mask = jnp.zeros((e, blk), jnp.bool_)
    top_vals = []
    top_rmaxs = []
    for k in range(NUM_SELECTS + 1):
        m = jnp.max(work, axis=0, keepdims=True)
        top_vals.append(m)
        if k < NUM_SELECTS:
            rsel = jnp.where(work == m, rf, -1.0)
            rmax = jnp.max(rsel, axis=0, keepdims=True)
            hit = rf == rmax
            top_rmaxs.append(rmax)
            sel_mask = sel_mask | hit
            work = jnp.where(hit, _NEG_BIG, work)

    maxv = top_vals[0]
    exps = [jnp.exp(v - maxv) for v in top_vals[:NUM_SELECTS]]
    denom = functools.reduce(jnp.add, exps)
    scores_out[...] = (jnp.concatenate(exps, axis=0) / denom).T
    idx_out[...] = (jnp.int32(e - 1)
                    - jnp.concatenate(top_rmaxs, axis=0).astype(jnp.int32)).T

    pe = jnp.where(sel_mask, jnp.exp(logits - maxv), 0.0)
    imp_blk = jnp.sum(pe / denom, axis=1, keepdims=True)

    t_in = top_vals[NUM_SELECTS]
    t_out = top_vals[NUM_SELECTS - 1]
    is_in = logits_noise > t_in
    thr = jnp.where(is_in, t_in, t_out)
    prob = _ndtr((logits_gate - thr) / noise_control)
    load_blk = jnp.sum(prob, axis=1, keepdims=True)

    @pl.when(i == 0)
    def _init():
        imp_out[...] = jnp.zeros_like(imp_out)
        load_out[...] = jnp.zeros_like(load_out)

    imp_out[...] += imp_blk
    load_out[...] += load_blk

    @pl.when(i == n - 1)
    def _finish():
        ef = jnp.float32(e)
        def cv2(v):
            mu = jnp.sum(v) / ef
            var = jnp.sum((v - mu) ** 2) / (ef - 1.0)
            return var / (mu * mu + 1e-10)
        loss = (cv2(imp_out[...]) + cv2(load_out[...])) * BLW
        loss_out[...] = jnp.broadcast_to(loss, (1, 1))


def kernel(x, W1, W2, Wn, noise):
    b, d = x.shape
    e = W1.shape[0]
    ns = NUM_SELECTS
    wc = jnp.concatenate([W1, Wn], axis=0)
    blk = min(2048, b)
    grid = (b // blk,)

    out_shapes = (
        jax.ShapeDtypeStruct((b, ns), jnp.int32),
        jax.ShapeDtypeStruct((b, ns), jnp.float32),
        jax.ShapeDtypeStruct((1, 1), jnp.float32),
        jax.ShapeDtypeStruct((e, 1), jnp.float32),
        jax.ShapeDtypeStruct((e, 1), jnp.float32),
    )
    in_specs = [
        pl.BlockSpec((blk, d), lambda i: (i, 0)),
        pl.BlockSpec((2 * e, d), lambda i: (0, 0)),
        pl.BlockSpec((e, e), lambda i: (0, 0)),
        pl.BlockSpec((blk, e), lambda i: (i, 0)),
    ]
    out_specs = (
        pl.BlockSpec((blk, ns), lambda i: (i, 0)),
        pl.BlockSpec((blk, ns), lambda i: (i, 0)),
        pl.BlockSpec((1, 1), lambda i: (0, 0)),
        pl.BlockSpec((e, 1), lambda i: (0, 0)),
        pl.BlockSpec((e, 1), lambda i: (0, 0)),
    )

    idx, scores, loss, load, imp = pl.pallas_call(
        _router_body,
        grid=grid,
        in_specs=in_specs,
        out_specs=out_specs,
        out_shape=out_shapes,
    )(x, wc, W2, noise)
    return (idx, scores, loss.reshape(()),
            load.reshape(e), imp.reshape(e))


# speedup vs baseline: 1.3894x; 1.3894x over previous
import functools

import jax
import jax.numpy as jnp
from jax.experimental import pallas as pl
from jax.experimental.pallas import tpu as pltpu

NUM_SELECTS = 8
NOISE_EPS = 0.01
BLW = 0.01
_NEG_BIG = -3.0e38
_INV_SQRT2 = 0.7071067811865476


def _ndtr(z):
    return 0.5 * (1.0 + jax.lax.erf(z * _INV_SQRT2))


def _router_body(x_ref, wc_ref, w2_ref, noise_t_ref,
                 idx_out, scores_out, loss_out, load_out, imp_out):
    i = pl.program_id(0)
    n = pl.num_programs(0)
    e, blk = noise_t_ref.shape

    mm = jax.lax.dot_general(
        wc_ref[...], x_ref[...], (((1,), (1,)), ((), ())),
        preferred_element_type=jnp.float32,
        precision=jax.lax.Precision.DEFAULT)
    h = jnp.tanh(mm[:e, :])
    noise_mm = mm[e:, :]
    logits_gate = jax.lax.dot_general(
        w2_ref[...], h, (((1,), (0,)), ((), ())),
        preferred_element_type=jnp.float32,
        precision=jax.lax.Precision.DEFAULT)
    noise_control = (jnp.maximum(noise_mm, 0.0)
                     + jnp.log1p(jnp.exp(-jnp.abs(noise_mm))) + NOISE_EPS)
    logits_noise = noise_t_ref[...] * noise_control
    logits = logits_gate + logits_noise

    rf = ((e - 1) - jax.lax.broadcasted_iota(jnp.int32, (e, blk), 0)
          ).astype(jnp.float32)
    work = logits
    sel_mask = jnp.zeros((e, blk), jnp.bool_)
    top_vals = []
    top_rmaxs = []
    for k in range(NUM_SELECTS + 1):
        m = jnp.max(work, axis=0, keepdims=True)
        top_vals.append(m)
        if k < NUM_SELECTS:
            rsel = jnp.where(work == m, rf, -1.0)
            rmax = jnp.max(rsel, axis=0, keepdims=True)
            hit = rf == rmax
            top_rmaxs.append(rmax)
            sel_mask = sel_mask | hit
            work = jnp.where(hit, _NEG_BIG, work)

    maxv = top_vals[0]
    exps = [jnp.exp(v - maxv) for v in top_vals[:NUM_SELECTS]]
    denom = functools.reduce(jnp.add, exps)
    scores_out[...] = jnp.concatenate(exps, axis=0) / denom
    idx_out[...] = (jnp.int32(e - 1)
                    - jnp.concatenate(top_rmaxs, axis=0).astype(jnp.int32))

    pe = jnp.where(sel_mask, jnp.exp(logits - maxv), 0.0)
    imp_blk = jnp.sum(pe / denom, axis=1, keepdims=True)

    t_in = top_vals[NUM_SELECTS]
    t_out = top_vals[NUM_SELECTS - 1]
    is_in = logits_noise > t_in
    thr = jnp.where(is_in, t_in, t_out)
    prob = _ndtr((logits_gate - thr) / noise_control)
    load_blk = jnp.sum(prob, axis=1, keepdims=True)

    @pl.when(i == 0)
    def _init():
        imp_out[...] = jnp.zeros_like(imp_out)
        load_out[...] = jnp.zeros_like(load_out)

    imp_out[...] += imp_blk
    load_out[...] += load_blk

    @pl.when(i == n - 1)
    def _finish():
        ef = jnp.float32(e)
        def cv2(v):
            mu = jnp.sum(v) / ef
            var = jnp.sum((v - mu) ** 2) / (ef - 1.0)
            return var / (mu * mu + 1e-10)
        loss = (cv2(imp_out[...]) + cv2(load_out[...])) * BLW
        loss_out[...] = jnp.broadcast_to(loss, (1, 1))


def kernel(x, W1, W2, Wn, noise):
    b, d = x.shape
    e = W1.shape[0]
    ns = NUM_SELECTS
    wc = jnp.concatenate([W1, Wn], axis=0)
    noise_t = noise.T
    blk = min(2048, b)
    grid = (b // blk,)

    out_shapes = (
        jax.ShapeDtypeStruct((ns, b), jnp.int32),
        jax.ShapeDtypeStruct((ns, b), jnp.float32),
        jax.ShapeDtypeStruct((1, 1), jnp.float32),
        jax.ShapeDtypeStruct((e, 1), jnp.float32),
        jax.ShapeDtypeStruct((e, 1), jnp.float32),
    )
    in_specs = [
        pl.BlockSpec((blk, d), lambda i: (i, 0)),
        pl.BlockSpec((2 * e, d), lambda i: (0, 0)),
        pl.BlockSpec((e, e), lambda i: (0, 0)),
        pl.BlockSpec((e, blk), lambda i: (0, i)),
    ]
    out_specs = (
        pl.BlockSpec((ns, blk), lambda i: (0, i)),
        pl.BlockSpec((ns, blk), lambda i: (0, i)),
        pl.BlockSpec((1, 1), lambda i: (0, 0)),
        pl.BlockSpec((e, 1), lambda i: (0, 0)),
        pl.BlockSpec((e, 1), lambda i: (0, 0)),
    )

    idx_t, scores_t, loss, load, imp = pl.pallas_call(
        _router_body,
        grid=grid,
        in_specs=in_specs,
        out_specs=out_specs,
        out_shape=out_shapes,
    )(x, wc, W2, noise_t)
    return (idx_t.T, scores_t.T, loss.reshape(()),
            load.reshape(e), imp.reshape(e))
